# baseline (speedup 1.0000x reference)
import functools

import jax
import jax.numpy as jnp
from jax import lax
from jax.experimental import pallas as pl
from jax.experimental.pallas import tpu as pltpu
from jax.experimental.pallas import tpu_sc as plsc

N = 10000
E = 320000
F_IN = 128
H = 256
C = 64

NC = 2
NS = 16
LANE = 16

NP = 10240
CE = 128
EP = 327680
NCHUNK = EP // CE
NCH = NCHUNK // NS
ROWS_PS = NP // NS
ZR = 64
EPS_D = EP // NS
SL = NP // NS

_mesh = functools.partial(
    plsc.VectorSubcoreMesh, core_axis_name="c", subcore_axis_name="s")


def _deg_call(edges):
    @functools.partial(
        pl.kernel,
        out_type=jax.ShapeDtypeStruct((2, NP), jnp.float32),
        mesh=_mesh(),
        compiler_params=pltpu.CompilerParams(needs_layout_passes=False),
        scratch_types=[
            pltpu.VMEM((EPS_D,), jnp.int32),
            pltpu.VMEM((NP,), jnp.float32),
            pltpu.VMEM((NS, SL), jnp.float32),
            pltpu.VMEM((SL,), jnp.float32),
            pltpu.VMEM_SHARED((NS, NP), jnp.float32),
            pltpu.SemaphoreType.DMA,
        ],
    )
    def deg_kernel(edges_hbm, out_hbm, idx_v, acc_v, tmp_v, sum_v, shared, sem):
        c = lax.axis_index("c")
        s = lax.axis_index("s")

        def zero(i, _):
            acc_v[pl.ds(i * LANE, LANE)] = jnp.zeros((LANE,), jnp.float32)
            return 0
        lax.fori_loop(0, NP // LANE, zero, 0)

        pltpu.sync_copy(edges_hbm.at[c, pl.ds(s * EPS_D, EPS_D)], idx_v)

        ones = jnp.ones((LANE,), jnp.float32)

        def accum(i, _):
            idx = idx_v[pl.ds(i * LANE, LANE)]
            plsc.addupdate_scatter(acc_v, [idx], ones)
            return 0
        lax.fori_loop(0, EPS_D // LANE, accum, 0)

        pltpu.sync_copy(acc_v, shared.at[s])
        plsc.subcore_barrier()

        cps = [
            pltpu.async_copy(shared.at[t, pl.ds(s * SL, SL)], tmp_v.at[t], sem)
            for t in range(NS)
        ]
        for cp in cps:
            cp.wait()

        def reduce(i, _):
            v = tmp_v[0, pl.ds(i * LANE, LANE)]
            for t in range(1, NS):
                v = v + tmp_v[t, pl.ds(i * LANE, LANE)]
            sum_v[pl.ds(i * LANE, LANE)] = v
            return 0
        lax.fori_loop(0, SL // LANE, reduce, 0)

        pltpu.sync_copy(sum_v, out_hbm.at[c, pl.ds(s * SL, SL)])

    return deg_kernel(edges)


def _make_agg(hw, npass):
    @functools.partial(
        pl.kernel,
        out_type=jax.ShapeDtypeStruct((2 * npass, NP, hw), jnp.float32),
        mesh=_mesh(),
        compiler_params=pltpu.CompilerParams(
            use_tc_tiling_on_sc=False, needs_layout_passes=False),
        scratch_types=[
            pltpu.VMEM((NCH // 2, CE), jnp.int32),
            pltpu.VMEM((NCH // 2, CE), jnp.int32),
            pltpu.VMEM((8, CE, hw), jnp.bfloat16),
            pltpu.VMEM((2, CE, hw), jnp.float32),
            pltpu.VMEM((ZR, hw), jnp.float32),
            pltpu.VMEM_SHARED((NP, hw), jnp.float32),
            pltpu.SemaphoreType.DMA((8,)),
            pltpu.SemaphoreType.DMA((2,)),
        ],
    )
    def agg_kernel(tbl_hbm, src_hbm, dst_hbm, out_hbm,
                   sidx, didx, rows, frows, zb, acc, gsems, ssems):
        c = lax.axis_index("c")
        s = lax.axis_index("s")

        def zrow(i, _):
            for k in range(hw // LANE):
                zb[i, pl.ds(k * LANE, LANE)] = jnp.zeros((LANE,), jnp.float32)
            return 0
        lax.fori_loop(0, ZR, zrow, 0)

        base = s * ROWS_PS

        def zcp(i, _):
            pltpu.sync_copy(zb, acc.at[pl.ds(base + i * ZR, ZR)])
            return 0
        lax.fori_loop(0, ROWS_PS // ZR, zcp, 0)

        def gstart(j, p):
            pltpu.async_copy(tbl_hbm.at[sidx.at[j]], rows.at[p], gsems.at[p])

        def gwait(j, p):
            pltpu.make_async_copy(
                tbl_hbm.at[sidx.at[j]], rows.at[p], gsems.at[p]).wait()

        def sstart(j, p):
            pltpu.async_copy(frows.at[p], acc.at[didx.at[j]], ssems.at[p],
                             add=True)

        def swait(j, p):
            pltpu.make_async_copy(
                frows.at[p], acc.at[didx.at[j]], ssems.at[p]).wait()

        def convert(gb, fb):
            def conv_rows(r4, _):
                for u in range(4):
                    r = r4 * 4 + u
                    for k in range(hw // 32):
                        v = rows[gb, r, pl.ds(k * 32, 32)]
                        lo, hi = plsc.unpack(
                            v, format=plsc.PackFormat.INTERLEAVED)
                        frows[fb, r, pl.ds(k * 32, LANE)] = lo
                        frows[fb, r, pl.ds(k * 32 + LANE, LANE)] = hi
                return 0
            lax.fori_loop(0, CE // 4, conv_rows, 0)

        def step(g, _):
            for b in range(8):
                j = g * 8 + b
                fb = b % 2

                @pl.when(j + 4 < NCH // 2)
                def _():
                    gstart(j + 4, (b + 4) % 8)

                @pl.when(j >= 2)
                def _():
                    swait(j - 2, fb)

                gwait(j, b)
                convert(b, fb)
                sstart(j, fb)
            return 0

        HCH = NCH // 2
        for p in range(npass):
            if p > 0:
                def zcp2(i, _):
                    pltpu.sync_copy(zb, acc.at[pl.ds(base + i * ZR, ZR)])
                    return 0
                lax.fori_loop(0, ROWS_PS // ZR, zcp2, 0)
            plsc.subcore_barrier()
            for half in range(2):
                off = s * NCH + half * HCH
                pltpu.sync_copy(
                    src_hbm.at[2 * p + c, pl.ds(off, HCH)], sidx)
                pltpu.sync_copy(dst_hbm.at[pl.ds(off, HCH)], didx)
                for jj in range(4):
                    gstart(jj, jj)
                lax.fori_loop(0, HCH // 8, step, 0)
                swait(HCH - 2, 0)
                swait(HCH - 1, 1)
            plsc.subcore_barrier()
            pltpu.sync_copy(acc.at[pl.ds(base, ROWS_PS)],
                            out_hbm.at[2 * p + c, pl.ds(base, ROWS_PS)])

    return agg_kernel


_agg64 = _make_agg(F_IN // 2, 1)
_agg1 = _make_agg(64, 2)
_agg32 = _make_agg(C // 2, 1)


RB = 1024
GRID = NP // RB


def _tc_scale_call(xp, degp):
    def body(x_ref, deg_ref, y_ref, ns_ref, nd_ref):
        ns = lax.rsqrt(jnp.maximum(deg_ref[0], 1.0))
        nd = lax.rsqrt(jnp.maximum(deg_ref[1], 1.0))
        ns_ref[...] = ns
        nd_ref[...] = nd
        y = (x_ref[...] * ns).astype(jnp.bfloat16)
        y_ref[0] = y[:, : F_IN // 2]
        y_ref[1] = y[:, F_IN // 2:]

    return pl.pallas_call(
        body,
        grid=(GRID,),
        in_specs=[
            pl.BlockSpec((RB, F_IN), lambda i: (i, 0)),
            pl.BlockSpec((2, RB, 1), lambda i: (0, i, 0)),
        ],
        out_specs=[
            pl.BlockSpec((2, RB, F_IN // 2), lambda i: (0, i, 0)),
            pl.BlockSpec((RB, 1), lambda i: (i, 0)),
            pl.BlockSpec((RB, 1), lambda i: (i, 0)),
        ],
        out_shape=[
            jax.ShapeDtypeStruct((2, NP, F_IN // 2), jnp.bfloat16),
            jax.ShapeDtypeStruct((NP, 1), jnp.float32),
            jax.ShapeDtypeStruct((NP, 1), jnp.float32),
        ],
    )(xp, degp)


def _tc_layer1_call(a0, nd, w0, b0, ns):
    hw_in = F_IN // 2
    Q = 64

    def body(a_ref, nd_ref, w_ref, b_ref, ns_ref, y_ref):
        ndv = nd_ref[...]
        z = (a_ref[0] * ndv) @ w_ref[: hw_in] + (a_ref[1] * ndv) @ w_ref[hw_in:]
        h = jnp.maximum(z + b_ref[...], 0.0)
        y = (h * ns_ref[...]).astype(jnp.bfloat16)
        for q in range(4):
            y_ref[q] = y[:, q * Q:(q + 1) * Q]

    return pl.pallas_call(
        body,
        grid=(GRID,),
        in_specs=[
            pl.BlockSpec((2, RB, hw_in), lambda i: (0, i, 0)),
            pl.BlockSpec((RB, 1), lambda i: (i, 0)),
            pl.BlockSpec((F_IN, H), lambda i: (0, 0)),
            pl.BlockSpec((1, H), lambda i: (0, 0)),
            pl.BlockSpec((RB, 1), lambda i: (i, 0)),
        ],
        out_specs=[pl.BlockSpec((4, RB, Q), lambda i: (0, i, 0))],
        out_shape=[jax.ShapeDtypeStruct((4, NP, Q), jnp.bfloat16)],
    )(a0, nd, w0, b0, ns)[0]


def _tc_layer2_call(a1, nd, w1, b1, ns, w2):
    Q = 64
    hw_out = C // 2

    def body(a_ref, nd_ref, w1_ref, b_ref, ns_ref, w2_ref, y_ref):
        ndv = nd_ref[...]
        z = ((a_ref[0] * ndv) @ w1_ref[:Q]
             + (a_ref[1] * ndv) @ w1_ref[Q:2 * Q]
             + (a_ref[2] * ndv) @ w1_ref[2 * Q:3 * Q]
             + (a_ref[3] * ndv) @ w1_ref[3 * Q:])
        h = jnp.maximum(z + b_ref[...], 0.0)
        y = ((h * ns_ref[...]) @ w2_ref[...]).astype(jnp.bfloat16)
        y_ref[0] = y[:, :hw_out]
        y_ref[1] = y[:, hw_out:]

    return pl.pallas_call(
        body,
        grid=(GRID,),
        in_specs=[
            pl.BlockSpec((4, RB, Q), lambda i: (0, i, 0)),
            pl.BlockSpec((RB, 1), lambda i: (i, 0)),
            pl.BlockSpec((H, H), lambda i: (0, 0)),
            pl.BlockSpec((1, H), lambda i: (0, 0)),
            pl.BlockSpec((RB, 1), lambda i: (i, 0)),
            pl.BlockSpec((H, C), lambda i: (0, 0)),
        ],
        out_specs=[pl.BlockSpec((2, RB, hw_out), lambda i: (0, i, 0))],
        out_shape=[jax.ShapeDtypeStruct((2, NP, hw_out), jnp.bfloat16)],
    )(a1, nd, w1, b1, ns, w2)[0]


def _tc_out_call(a2, nd, b2):
    def body(a_ref, nd_ref, b_ref, o_ref):
        o = jnp.concatenate([a_ref[0], a_ref[1]], axis=1) * nd_ref[...]
        o = o + b_ref[...]
        m = jnp.max(o, axis=1, keepdims=True)
        e = jnp.exp(o - m)
        o_ref[...] = (o - m) - jnp.log(jnp.sum(e, axis=1, keepdims=True))

    return pl.pallas_call(
        body,
        grid=(GRID,),
        in_specs=[
            pl.BlockSpec((2, RB, C // 2), lambda i: (0, i, 0)),
            pl.BlockSpec((RB, 1), lambda i: (i, 0)),
            pl.BlockSpec((1, C), lambda i: (0, 0)),
        ],
        out_specs=pl.BlockSpec((RB, C), lambda i: (i, 0)),
        out_shape=jax.ShapeDtypeStruct((NP, C), jnp.float32),
    )(a2, nd, b2)


def _perm_half(hw):
    p = []
    for g in range(hw // 32):
        b = g * 32
        p += [b + 2 * i for i in range(16)]
        p += [b + 2 * i + 1 for i in range(16)]
    return p


_P64 = _perm_half(64)
_P128 = _P64 + [64 + i for i in _P64]
_P256 = [64 * q + i for q in range(4) for i in _P64]
_P32 = _perm_half(32)
_PC = _P32 + [32 + i for i in _P32]
_PC_INV = sorted(range(len(_PC)), key=_PC.__getitem__)


def kernel(x, edge_index, W0, b0, W1, b1, W2, b2):
    src = edge_index[0]
    dst = edge_index[1]
    padi = jnp.full((EP - E,), NP - 1, jnp.int32)
    srcp = jnp.concatenate([src, padi])
    dstp = jnp.concatenate([dst, padi])
    edges = jnp.stack([srcp, dstp])
    src2a = jnp.stack([srcp, srcp + NP]).reshape(2, NCHUNK, CE)
    src4 = jnp.stack([srcp, srcp + NP, srcp + 2 * NP,
                      srcp + 3 * NP]).reshape(4, NCHUNK, CE)
    dst2 = dstp.reshape(NCHUNK, CE)

    xp = jnp.concatenate([x, jnp.zeros((NP - N, F_IN), x.dtype)])

    W0p = W0[jnp.array(_P128)]
    W1p = W1[jnp.array(_P256)]
    W2p = W2[:, jnp.array(_PC_INV)]

    deg = _deg_call(edges)
    y0, ns, nd = _tc_scale_call(xp, deg.reshape(2, NP, 1))
    a0 = _agg64(y0.reshape(2 * NP, F_IN // 2), src2a, dst2)
    y1 = _tc_layer1_call(a0, nd, W0p, b0.reshape(1, H), ns)
    a1 = _agg1(y1.reshape(4 * NP, 64), src4, dst2)
    y2 = _tc_layer2_call(a1, nd, W1p, b1.reshape(1, H), ns, W2p)
    a2 = _agg32(y2.reshape(2 * NP, C // 2), src2a, dst2)
    out = _tc_out_call(a2, nd, b2.reshape(1, C))
    return out[:N]


# speedup vs baseline: 1.0178x; 1.0178x over previous
import functools

import jax
import jax.numpy as jnp
from jax import lax
from jax.experimental import pallas as pl
from jax.experimental.pallas import tpu as pltpu
from jax.experimental.pallas import tpu_sc as plsc

N = 10000
E = 320000
F_IN = 128
H = 256
C = 64

NC = 2
NS = 16
LANE = 16

NP = 10240
CE = 128
EP = 327680
NCHUNK = EP // CE
NCH = NCHUNK // NS
ROWS_PS = NP // NS
ZR = 64
EPS_D = EP // NS
SL = NP // NS

_mesh = functools.partial(
    plsc.VectorSubcoreMesh, core_axis_name="c", subcore_axis_name="s")


def _deg_call(edges):
    @functools.partial(
        pl.kernel,
        out_type=jax.ShapeDtypeStruct((2, NP), jnp.float32),
        mesh=_mesh(),
        compiler_params=pltpu.CompilerParams(needs_layout_passes=False),
        scratch_types=[
            pltpu.VMEM((EPS_D,), jnp.int32),
            pltpu.VMEM((NP,), jnp.float32),
            pltpu.VMEM((NS, SL), jnp.float32),
            pltpu.VMEM((SL,), jnp.float32),
            pltpu.VMEM_SHARED((NS, NP), jnp.float32),
            pltpu.SemaphoreType.DMA,
        ],
    )
    def deg_kernel(edges_hbm, out_hbm, idx_v, acc_v, tmp_v, sum_v, shared, sem):
        c = lax.axis_index("c")
        s = lax.axis_index("s")

        def zero(i, _):
            acc_v[pl.ds(i * LANE, LANE)] = jnp.zeros((LANE,), jnp.float32)
            return 0
        lax.fori_loop(0, NP // LANE, zero, 0)

        pltpu.sync_copy(edges_hbm.at[c, pl.ds(s * EPS_D, EPS_D)], idx_v)

        ones = jnp.ones((LANE,), jnp.float32)

        def accum(i, _):
            idx = idx_v[pl.ds(i * LANE, LANE)]
            plsc.addupdate_scatter(acc_v, [idx], ones)
            return 0
        lax.fori_loop(0, EPS_D // LANE, accum, 0)

        pltpu.sync_copy(acc_v, shared.at[s])
        plsc.subcore_barrier()

        cps = [
            pltpu.async_copy(shared.at[t, pl.ds(s * SL, SL)], tmp_v.at[t], sem)
            for t in range(NS)
        ]
        for cp in cps:
            cp.wait()

        def reduce(i, _):
            v = tmp_v[0, pl.ds(i * LANE, LANE)]
            for t in range(1, NS):
                v = v + tmp_v[t, pl.ds(i * LANE, LANE)]
            sum_v[pl.ds(i * LANE, LANE)] = v
            return 0
        lax.fori_loop(0, SL // LANE, reduce, 0)

        pltpu.sync_copy(sum_v, out_hbm.at[c, pl.ds(s * SL, SL)])

    return deg_kernel(edges)


def _make_agg(hw, npass):
    @functools.partial(
        pl.kernel,
        out_type=jax.ShapeDtypeStruct((2 * npass, NP, hw), jnp.float32),
        mesh=_mesh(),
        compiler_params=pltpu.CompilerParams(
            use_tc_tiling_on_sc=False, needs_layout_passes=False),
        scratch_types=[
            pltpu.VMEM((NCH, CE), jnp.int32),
            pltpu.VMEM((NCH, CE), jnp.int32),
            pltpu.VMEM((4, CE, hw), jnp.bfloat16),
            pltpu.VMEM((2, CE, hw), jnp.float32),
            pltpu.VMEM((ZR, hw), jnp.float32),
            pltpu.VMEM_SHARED((NP, hw), jnp.float32),
            pltpu.SemaphoreType.DMA((4,)),
            pltpu.SemaphoreType.DMA((2,)),
        ],
    )
    def agg_kernel(tbl_hbm, src_hbm, dst_hbm, out_hbm,
                   sidx, didx, rows, frows, zb, acc, gsems, ssems):
        c = lax.axis_index("c")
        s = lax.axis_index("s")

        def zrow(i, _):
            for k in range(hw // LANE):
                zb[i, pl.ds(k * LANE, LANE)] = jnp.zeros((LANE,), jnp.float32)
            return 0
        lax.fori_loop(0, ZR, zrow, 0)

        base = s * ROWS_PS

        def zcp(i, _):
            pltpu.sync_copy(zb, acc.at[pl.ds(base + i * ZR, ZR)])
            return 0
        lax.fori_loop(0, ROWS_PS // ZR, zcp, 0)

        pltpu.sync_copy(dst_hbm.at[pl.ds(s * NCH, NCH)], didx)

        def gstart(j, p):
            pltpu.async_copy(tbl_hbm.at[sidx.at[j]], rows.at[p], gsems.at[p])

        def gwait(j, p):
            pltpu.make_async_copy(
                tbl_hbm.at[sidx.at[j]], rows.at[p], gsems.at[p]).wait()

        def sstart(j, p):
            pltpu.async_copy(frows.at[p], acc.at[didx.at[j]], ssems.at[p],
                             add=True)

        def swait(j, p):
            pltpu.make_async_copy(
                frows.at[p], acc.at[didx.at[j]], ssems.at[p]).wait()

        def convert(gb, fb):
            def conv_row(r, _):
                for k in range(hw // 32):
                    v = rows[gb, r, pl.ds(k * 32, 32)]
                    lo, hi = plsc.unpack(
                        v, format=plsc.PackFormat.INTERLEAVED)
                    frows[fb, r, pl.ds(k * 32, LANE)] = lo
                    frows[fb, r, pl.ds(k * 32 + LANE, LANE)] = hi
                return 0
            lax.fori_loop(0, CE, conv_row, 0)

        def step(g, _):
            for b in range(4):
                j = g * 4 + b
                fb = b % 2

                @pl.when(j + 2 < NCH)
                def _():
                    gstart(j + 2, (b + 2) % 4)

                @pl.when(j >= 2)
                def _():
                    swait(j - 2, fb)

                gwait(j, b)
                convert(b, fb)
                sstart(j, fb)
            return 0

        for p in range(npass):
            if p > 0:
                def zcp2(i, _):
                    pltpu.sync_copy(zb, acc.at[pl.ds(base + i * ZR, ZR)])
                    return 0
                lax.fori_loop(0, ROWS_PS // ZR, zcp2, 0)
            pltpu.sync_copy(src_hbm.at[2 * p + c, pl.ds(s * NCH, NCH)], sidx)
            plsc.subcore_barrier()
            gstart(0, 0)
            gstart(1, 1)
            lax.fori_loop(0, NCH // 4, step, 0)
            swait(NCH - 2, 0)
            swait(NCH - 1, 1)
            plsc.subcore_barrier()
            pltpu.sync_copy(acc.at[pl.ds(base, ROWS_PS)],
                            out_hbm.at[2 * p + c, pl.ds(base, ROWS_PS)])

    return agg_kernel


_agg64 = _make_agg(F_IN // 2, 1)
_agg1 = _make_agg(64, 2)
_agg32 = _make_agg(C // 2, 1)


RB = 1024
GRID = NP // RB


def _tc_scale_call(xp, degp):
    def body(x_ref, deg_ref, y_ref, ns_ref, nd_ref):
        ns = lax.rsqrt(jnp.maximum(deg_ref[0], 1.0))
        nd = lax.rsqrt(jnp.maximum(deg_ref[1], 1.0))
        ns_ref[...] = ns
        nd_ref[...] = nd
        y = (x_ref[...] * ns).astype(jnp.bfloat16)
        y_ref[0] = y[:, : F_IN // 2]
        y_ref[1] = y[:, F_IN // 2:]

    return pl.pallas_call(
        body,
        grid=(GRID,),
        in_specs=[
            pl.BlockSpec((RB, F_IN), lambda i: (i, 0)),
            pl.BlockSpec((2, RB, 1), lambda i: (0, i, 0)),
        ],
        out_specs=[
            pl.BlockSpec((2, RB, F_IN // 2), lambda i: (0, i, 0)),
            pl.BlockSpec((RB, 1), lambda i: (i, 0)),
            pl.BlockSpec((RB, 1), lambda i: (i, 0)),
        ],
        out_shape=[
            jax.ShapeDtypeStruct((2, NP, F_IN // 2), jnp.bfloat16),
            jax.ShapeDtypeStruct((NP, 1), jnp.float32),
            jax.ShapeDtypeStruct((NP, 1), jnp.float32),
        ],
    )(xp, degp)


def _tc_layer1_call(a0, nd, w0, b0, ns):
    hw_in = F_IN // 2
    Q = 64

    def body(a_ref, nd_ref, w_ref, b_ref, ns_ref, y_ref):
        ndv = nd_ref[...]
        z = (a_ref[0] * ndv) @ w_ref[: hw_in] + (a_ref[1] * ndv) @ w_ref[hw_in:]
        h = jnp.maximum(z + b_ref[...], 0.0)
        y = (h * ns_ref[...]).astype(jnp.bfloat16)
        for q in range(4):
            y_ref[q] = y[:, q * Q:(q + 1) * Q]

    return pl.pallas_call(
        body,
        grid=(GRID,),
        in_specs=[
            pl.BlockSpec((2, RB, hw_in), lambda i: (0, i, 0)),
            pl.BlockSpec((RB, 1), lambda i: (i, 0)),
            pl.BlockSpec((F_IN, H), lambda i: (0, 0)),
            pl.BlockSpec((1, H), lambda i: (0, 0)),
            pl.BlockSpec((RB, 1), lambda i: (i, 0)),
        ],
        out_specs=[pl.BlockSpec((4, RB, Q), lambda i: (0, i, 0))],
        out_shape=[jax.ShapeDtypeStruct((4, NP, Q), jnp.bfloat16)],
    )(a0, nd, w0, b0, ns)[0]


def _tc_layer2_call(a1, nd, w1, b1, ns, w2):
    Q = 64
    hw_out = C // 2

    def body(a_ref, nd_ref, w1_ref, b_ref, ns_ref, w2_ref, y_ref):
        ndv = nd_ref[...]
        z = ((a_ref[0] * ndv) @ w1_ref[:Q]
             + (a_ref[1] * ndv) @ w1_ref[Q:2 * Q]
             + (a_ref[2] * ndv) @ w1_ref[2 * Q:3 * Q]
             + (a_ref[3] * ndv) @ w1_ref[3 * Q:])
        h = jnp.maximum(z + b_ref[...], 0.0)
        y = ((h * ns_ref[...]) @ w2_ref[...]).astype(jnp.bfloat16)
        y_ref[0] = y[:, :hw_out]
        y_ref[1] = y[:, hw_out:]

    return pl.pallas_call(
        body,
        grid=(GRID,),
        in_specs=[
            pl.BlockSpec((4, RB, Q), lambda i: (0, i, 0)),
            pl.BlockSpec((RB, 1), lambda i: (i, 0)),
            pl.BlockSpec((H, H), lambda i: (0, 0)),
            pl.BlockSpec((1, H), lambda i: (0, 0)),
            pl.BlockSpec((RB, 1), lambda i: (i, 0)),
            pl.BlockSpec((H, C), lambda i: (0, 0)),
        ],
        out_specs=[pl.BlockSpec((2, RB, hw_out), lambda i: (0, i, 0))],
        out_shape=[jax.ShapeDtypeStruct((2, NP, hw_out), jnp.bfloat16)],
    )(a1, nd, w1, b1, ns, w2)[0]


def _tc_out_call(a2, nd, b2):
    def body(a_ref, nd_ref, b_ref, o_ref):
        o = jnp.concatenate([a_ref[0], a_ref[1]], axis=1) * nd_ref[...]
        o = o + b_ref[...]
        m = jnp.max(o, axis=1, keepdims=True)
        e = jnp.exp(o - m)
        o_ref[...] = (o - m) - jnp.log(jnp.sum(e, axis=1, keepdims=True))

    return pl.pallas_call(
        body,
        grid=(GRID,),
        in_specs=[
            pl.BlockSpec((2, RB, C // 2), lambda i: (0, i, 0)),
            pl.BlockSpec((RB, 1), lambda i: (i, 0)),
            pl.BlockSpec((1, C), lambda i: (0, 0)),
        ],
        out_specs=pl.BlockSpec((RB, C), lambda i: (i, 0)),
        out_shape=jax.ShapeDtypeStruct((NP, C), jnp.float32),
    )(a2, nd, b2)


def _perm_half(hw):
    p = []
    for g in range(hw // 32):
        b = g * 32
        p += [b + 2 * i for i in range(16)]
        p += [b + 2 * i + 1 for i in range(16)]
    return p


_P64 = _perm_half(64)
_P128 = _P64 + [64 + i for i in _P64]
_P256 = [64 * q + i for q in range(4) for i in _P64]
_P32 = _perm_half(32)
_PC = _P32 + [32 + i for i in _P32]
_PC_INV = sorted(range(len(_PC)), key=_PC.__getitem__)


def kernel(x, edge_index, W0, b0, W1, b1, W2, b2):
    src = edge_index[0]
    dst = edge_index[1]
    padi = jnp.full((EP - E,), NP - 1, jnp.int32)
    srcp = jnp.concatenate([src, padi])
    dstp = jnp.concatenate([dst, padi])
    edges = jnp.stack([srcp, dstp])
    src2a = jnp.stack([srcp, srcp + NP]).reshape(2, NCHUNK, CE)
    src4 = jnp.stack([srcp, srcp + NP, srcp + 2 * NP,
                      srcp + 3 * NP]).reshape(4, NCHUNK, CE)
    dst2 = dstp.reshape(NCHUNK, CE)

    xp = jnp.concatenate([x, jnp.zeros((NP - N, F_IN), x.dtype)])

    W0p = W0[jnp.array(_P128)]
    W1p = W1[jnp.array(_P256)]
    W2p = W2[:, jnp.array(_PC_INV)]

    deg = _deg_call(edges)
    y0, ns, nd = _tc_scale_call(xp, deg.reshape(2, NP, 1))
    a0 = _agg64(y0.reshape(2 * NP, F_IN // 2), src2a, dst2)
    y1 = _tc_layer1_call(a0, nd, W0p, b0.reshape(1, H), ns)
    a1 = _agg1(y1.reshape(4 * NP, 64), src4, dst2)
    y2 = _tc_layer2_call(a1, nd, W1p, b1.reshape(1, H), ns, W2p)
    a2 = _agg32(y2.reshape(2 * NP, C // 2), src2a, dst2)
    out = _tc_out_call(a2, nd, b2.reshape(1, C))
    return out[:N]
